# baseline (speedup 1.0000x reference)
import functools

import jax
import jax.numpy as jnp
from jax import lax
from jax.experimental import pallas as pl
from jax.experimental.pallas import tpu as pltpu
from jax.experimental.pallas import tpu_sc as plsc

N = 100000
D = 128
S = 256
L = 16
NC = 2
NS = 16
NW = NC * NS
SEG_PER_W = S // NW
R = 64
KV = D // L


def _body(x_hbm, batch_hbm, wb_hbm, out_hbm, batch_v, xbuf_v, wb_v, stage_v):
    wid = lax.axis_index("c") * NS + lax.axis_index("s")

    pltpu.sync_copy(wb_hbm, wb_v)
    pltpu.sync_copy(batch_hbm, batch_v)

    w = [wb_v[0, pl.ds(k * L, L)] for k in range(KV)]
    bvec = wb_v[1, pl.ds(0, L)]

    t = wid * SEG_PER_W + lax.iota(jnp.int32, L)
    lo0 = jnp.zeros((L,), jnp.int32)
    hi0 = jnp.full((L,), N, jnp.int32)

    def sbody(_, c):
        lo, hi = c
        act = lo < hi
        mid = lax.shift_right_logical(lo + hi, 1)
        vals = plsc.load_gather(batch_v, [jnp.minimum(mid, N - 1)])
        less = vals < t
        lo = jnp.where(act & less, mid + 1, lo)
        hi = jnp.where(act & (~less), mid, hi)
        return lo, hi

    lo, _ = lax.fori_loop(0, 17, sbody, (lo0, hi0))

    zero = jnp.zeros((L,), jnp.float32)
    ninf = jnp.full((L,), -jnp.inf, jnp.float32)

    for j in range(SEG_PER_W):
        seg_lo = lo[j]
        seg_hi = lo[j + 1]

        def cond_fn(c):
            return c[0] < seg_hi

        def chunk(c):
            r = c[0]
            dstart = pl.multiple_of(jnp.minimum(r & ~7, N - R), 8)
            pltpu.sync_copy(x_hbm.at[pl.ds(dstart, R)], xbuf_v)
            off = r - dstart
            nrows = jnp.minimum(R - off, seg_hi - r)

            def row_work(ri, c2):
                xs = [xbuf_v[ri, pl.ds(k * L, L)] for k in range(KV)]
                acc = xs[0] * w[0]
                for k in range(1, KV):
                    acc = acc + xs[k] * w[k]
                d = jnp.sum(acc)
                zv = jnp.full((L,), d, jnp.float32) + bvec
                sv = 1.0 / (1.0 + jnp.exp(-zv))
                sums = tuple(c2[k] + sv * xs[k] for k in range(KV))
                maxs = tuple(jnp.maximum(c2[KV + k], xs[k]) for k in range(KV))
                return sums + maxs

            def quad(q, c2):
                base = off + q * 4
                for u in range(4):
                    c2 = row_work(base + u, c2)
                return c2

            n4 = nrows & ~3
            res = lax.fori_loop(0, n4 >> 2, quad, c[1:])
            res = lax.fori_loop(n4, nrows, lambda i, c2: row_work(off + i, c2), res)
            return (r + nrows,) + res

        fin = lax.while_loop(cond_fn, chunk, (seg_lo,) + (zero,) * KV + (ninf,) * KV)
        for k in range(KV):
            stage_v[j, pl.ds(k * L, L)] = fin[1 + k]
            stage_v[j, pl.ds(D + k * L, L)] = fin[1 + KV + k]

    pltpu.sync_copy(stage_v, out_hbm.at[pl.ds(wid * SEG_PER_W, SEG_PER_W)])


@jax.jit
def kernel(x, batch, W, b):
    batch32 = batch.astype(jnp.int32)
    wb = jnp.concatenate(
        [W.astype(jnp.float32),
         jnp.broadcast_to(b.astype(jnp.float32).reshape(1, 1), (1, D))], axis=0)
    mesh = plsc.VectorSubcoreMesh(core_axis_name="c", subcore_axis_name="s")
    fn = functools.partial(
        pl.kernel,
        out_type=jax.ShapeDtypeStruct((S, 2 * D), jnp.float32),
        mesh=mesh,
        compiler_params=pltpu.CompilerParams(needs_layout_passes=False),
        scratch_types=[
            pltpu.VMEM((N,), jnp.int32),
            pltpu.VMEM((R, D), jnp.float32),
            pltpu.VMEM((2, D), jnp.float32),
            pltpu.VMEM((SEG_PER_W, 2 * D), jnp.float32),
        ],
    )(_body)
    return fn(x, batch32, wb)


# speedup vs baseline: 1.1686x; 1.1686x over previous
import functools

import jax
import jax.numpy as jnp
from jax import lax
from jax.experimental import pallas as pl
from jax.experimental.pallas import tpu as pltpu
from jax.experimental.pallas import tpu_sc as plsc

N = 100000
D = 128
S = 256
L = 16
NC = 2
NS = 16
NW = NC * NS
SEG_PER_W = S // NW
R = 64
KV = D // L


def _body(x_hbm, batch_hbm, wb_hbm, out_hbm, batch_v, xbuf_v, wb_v, stage_v):
    wid = lax.axis_index("c") * NS + lax.axis_index("s")

    pltpu.sync_copy(wb_hbm, wb_v)
    pltpu.sync_copy(batch_hbm, batch_v)

    w = [wb_v[0, pl.ds(k * L, L)] for k in range(KV)]
    bvec = wb_v[1, pl.ds(0, L)]

    t = wid * SEG_PER_W + lax.iota(jnp.int32, L)
    lo0 = jnp.zeros((L,), jnp.int32)
    hi0 = jnp.full((L,), N, jnp.int32)

    def sbody(_, c):
        lo, hi = c
        act = lo < hi
        mid = lax.shift_right_logical(lo + hi, 1)
        vals = plsc.load_gather(batch_v, [jnp.minimum(mid, N - 1)])
        less = vals < t
        lo = jnp.where(act & less, mid + 1, lo)
        hi = jnp.where(act & (~less), mid, hi)
        return lo, hi

    lo, _ = lax.fori_loop(0, 17, sbody, (lo0, hi0))

    zero = jnp.zeros((L,), jnp.float32)
    ninf = jnp.full((L,), -jnp.inf, jnp.float32)

    for j in range(SEG_PER_W):
        seg_lo = lo[j]
        seg_hi = lo[j + 1]

        def cond_fn(c):
            return c[0] < seg_hi

        def chunk(c):
            r = c[0]
            dstart = pl.multiple_of(jnp.minimum(r & ~7, N - R), 8)
            pltpu.sync_copy(x_hbm.at[pl.ds(dstart, R)], xbuf_v)
            off = r - dstart
            nrows = jnp.minimum(R - off, seg_hi - r)

            def row_work(ri, c2):
                xs = [xbuf_v[ri, pl.ds(k * L, L)] for k in range(KV)]
                acc = xs[0] * w[0]
                for k in range(1, KV):
                    acc = acc + xs[k] * w[k]
                d = jnp.sum(acc)
                zv = jnp.full((L,), d, jnp.float32) + bvec
                sv = 1.0 / (1.0 + jnp.exp(-zv))
                sums = tuple(c2[k] + sv * xs[k] for k in range(KV))
                maxs = tuple(jnp.maximum(c2[KV + k], xs[k]) for k in range(KV))
                return sums + maxs

            def pair(q, c2):
                base = off + q * 2
                for u in range(2):
                    c2 = row_work(base + u, c2)
                return c2

            n2 = nrows & ~1
            res = lax.fori_loop(0, n2 >> 1, pair, c[1:])
            res = lax.fori_loop(n2, nrows, lambda i, c2: row_work(off + i, c2), res)
            return (r + nrows,) + res

        fin = lax.while_loop(cond_fn, chunk, (seg_lo,) + (zero,) * KV + (ninf,) * KV)
        for k in range(KV):
            stage_v[j, pl.ds(k * L, L)] = fin[1 + k]
            stage_v[j, pl.ds(D + k * L, L)] = fin[1 + KV + k]

    pltpu.sync_copy(stage_v, out_hbm.at[pl.ds(wid * SEG_PER_W, SEG_PER_W)])


@jax.jit
def kernel(x, batch, W, b):
    batch32 = batch.astype(jnp.int32)
    wb = jnp.concatenate(
        [W.astype(jnp.float32),
         jnp.broadcast_to(b.astype(jnp.float32).reshape(1, 1), (1, D))], axis=0)
    mesh = plsc.VectorSubcoreMesh(core_axis_name="c", subcore_axis_name="s")
    fn = functools.partial(
        pl.kernel,
        out_type=jax.ShapeDtypeStruct((S, 2 * D), jnp.float32),
        mesh=mesh,
        compiler_params=pltpu.CompilerParams(needs_layout_passes=False),
        scratch_types=[
            pltpu.VMEM((N,), jnp.int32),
            pltpu.VMEM((R, D), jnp.float32),
            pltpu.VMEM((2, D), jnp.float32),
            pltpu.VMEM((SEG_PER_W, 2 * D), jnp.float32),
        ],
    )(_body)
    return fn(x, batch32, wb)


# speedup vs baseline: 2.0402x; 1.7459x over previous
import functools

import jax
import jax.numpy as jnp
from jax import lax
from jax.experimental import pallas as pl
from jax.experimental.pallas import tpu as pltpu
from jax.experimental.pallas import tpu_sc as plsc

N = 100000
D = 128
S = 256
L = 16
NC = 2
NS = 16
NW = NC * NS
SEG_PER_W = S // NW
R = 64
KV = D // L


def _body(x_hbm, batch_hbm, wb_hbm, out_hbm, batch_v, xbuf_v, wb_v, stage_v,
          bnd_s, sem):
    wid = lax.axis_index("c") * NS + lax.axis_index("s")

    pltpu.sync_copy(wb_hbm, wb_v)
    pltpu.sync_copy(batch_hbm, batch_v)

    w = [wb_v[0, pl.ds(k * L, L)] for k in range(KV)]
    bvec = wb_v[1, pl.ds(0, L)]

    t = wid * SEG_PER_W + lax.iota(jnp.int32, L)
    lo0 = jnp.zeros((L,), jnp.int32)
    hi0 = jnp.full((L,), N, jnp.int32)

    def sbody(_, c):
        lo, hi = c
        act = lo < hi
        mid = lax.shift_right_logical(lo + hi, 1)
        vals = plsc.load_gather(batch_v, [jnp.minimum(mid, N - 1)])
        less = vals < t
        lo = jnp.where(act & less, mid + 1, lo)
        hi = jnp.where(act & (~less), mid, hi)
        return lo, hi

    lo, _ = lax.fori_loop(0, 17, sbody, (lo0, hi0))
    for i in range(SEG_PER_W + 1):
        bnd_s[i] = lo[i]

    zero = jnp.zeros((L,), jnp.float32)
    ninf = jnp.full((L,), -jnp.inf, jnp.float32)

    for j in range(SEG_PER_W):
        for k in range(KV):
            stage_v[j, pl.ds(k * L, L)] = zero
            stage_v[j, pl.ds(D + k * L, L)] = ninf

    r0 = lo[0]
    range_end = lo[SEG_PER_W]
    dbase0 = pl.multiple_of(jnp.minimum(r0 & ~7, N - R), 8)

    @pl.when(r0 < range_end)
    def _prologue():
        pltpu.sync_copy(x_hbm.at[pl.ds(dbase0, R)], xbuf_v.at[pl.ds(0, R)])

    def wcond(c):
        return c[0] < range_end

    def wbody(c):
        r, j, p, dbase, dp = c[0], c[1], c[2], c[3], c[4]
        sums = c[5:5 + KV]
        maxs = c[5 + KV:5 + 2 * KV]
        xsp = c[5 + 2 * KV:5 + 3 * KV]

        seg_end = bnd_s[j + 1]
        chunk_end = dbase + R
        stop = jnp.minimum(seg_end, chunk_end)
        need_next = (stop == chunk_end) & (stop < range_end)
        ndbase = pl.multiple_of(jnp.minimum(stop, N - R), 8)
        nxt = 1 - p

        @pl.when(need_next)
        def _prefetch():
            pltpu.async_copy(
                x_hbm.at[pl.ds(ndbase, R)],
                xbuf_v.at[pl.ds(pl.multiple_of(nxt * R, 8), R)], sem)

        prow = p * R + (r - dbase)

        def row(i, c2):
            sums = c2[:KV]
            maxs = c2[KV:2 * KV]
            xsp = c2[2 * KV:3 * KV]
            dp = c2[3 * KV]
            ri = prow + i
            xs = [xbuf_v[ri, pl.ds(k * L, L)] for k in range(KV)]
            acc = xs[0] * w[0]
            for k in range(1, KV):
                acc = acc + xs[k] * w[k]
            d = jnp.sum(acc)
            zv = jnp.full((L,), dp, jnp.float32) + bvec
            sv = 1.0 / (1.0 + jnp.exp(-zv))
            nsums = tuple(sums[k] + sv * xsp[k] for k in range(KV))
            nmaxs = tuple(jnp.maximum(maxs[k], xs[k]) for k in range(KV))
            return nsums + nmaxs + tuple(xs) + (d,)

        st = lax.fori_loop(0, stop - r, row, sums + maxs + xsp + (dp,))
        sums = st[:KV]
        maxs = st[KV:2 * KV]
        xsp = st[2 * KV:3 * KV]
        dp = st[3 * KV]

        def do_flush(op):
            sums, maxs, xsp, dp, j = op
            zv = jnp.full((L,), dp, jnp.float32) + bvec
            sv = 1.0 / (1.0 + jnp.exp(-zv))
            for k in range(KV):
                stage_v[j, pl.ds(k * L, L)] = sums[k] + sv * xsp[k]
                stage_v[j, pl.ds(D + k * L, L)] = maxs[k]
            return ((zero,) * KV, (ninf,) * KV, (zero,) * KV,
                    jnp.float32(0.0), j + 1)

        sums, maxs, xsp, dp, j = lax.cond(
            stop == seg_end, do_flush, lambda op: op,
            (tuple(sums), tuple(maxs), tuple(xsp), dp, j))

        @pl.when(need_next)
        def _flip_wait():
            pltpu.make_async_copy(
                x_hbm.at[pl.ds(0, R)], xbuf_v.at[pl.ds(0, R)], sem).wait()

        p = jnp.where(need_next, nxt, p)
        dbase = jnp.where(need_next, ndbase, dbase)
        return (stop, j, p, dbase, dp) + tuple(sums) + tuple(maxs) + tuple(xsp)

    init = ((r0, jnp.int32(0), jnp.int32(0), dbase0, jnp.float32(0.0))
            + (zero,) * KV + (ninf,) * KV + (zero,) * KV)
    lax.while_loop(wcond, wbody, init)

    pltpu.sync_copy(stage_v, out_hbm.at[pl.ds(wid * SEG_PER_W, SEG_PER_W)])


@jax.jit
def kernel(x, batch, W, b):
    batch32 = batch.astype(jnp.int32)
    wb = jnp.concatenate(
        [W.astype(jnp.float32),
         jnp.broadcast_to(b.astype(jnp.float32).reshape(1, 1), (1, D))], axis=0)
    mesh = plsc.VectorSubcoreMesh(core_axis_name="c", subcore_axis_name="s")
    fn = functools.partial(
        pl.kernel,
        out_type=jax.ShapeDtypeStruct((S, 2 * D), jnp.float32),
        mesh=mesh,
        compiler_params=pltpu.CompilerParams(needs_layout_passes=False),
        scratch_types=[
            pltpu.VMEM((N,), jnp.int32),
            pltpu.VMEM((2 * R, D), jnp.float32),
            pltpu.VMEM((2, D), jnp.float32),
            pltpu.VMEM((SEG_PER_W, 2 * D), jnp.float32),
            pltpu.SMEM((L,), jnp.int32),
            pltpu.SemaphoreType.DMA,
        ],
    )(_body)
    return fn(x, batch32, wb)
